# Initial kernel scaffold; baseline (speedup 1.0000x reference)
import jax
import jax.numpy as jnp
from jax.experimental import pallas as pl


def kernel(boxes, scores, img_h, img_w):
    raise NotImplementedError("write your pallas kernel here")


# speedup vs baseline: 349.1698x; 349.1698x over previous
import jax
import jax.numpy as jnp
from jax import lax
from jax.experimental import pallas as pl
from jax.experimental.pallas import tpu as pltpu

N_BOX = 20000
N_PAD = 20480
K_SEL = 2000
K_PAD = 2048
POST = 1000
OUT_ROWS = 1024
IOU_THR = 0.7
MIN_SIZE = 16.0
ROW_BLK = 256


def _mask_scores_kernel(hw_ref, bx_ref, sc_ref, out_ref):
    h = hw_ref[0]
    w = hw_ref[1]
    cx1 = jnp.clip(bx_ref[0:1, :], 0.0, w)
    cy1 = jnp.clip(bx_ref[1:2, :], 0.0, h)
    cx2 = jnp.clip(bx_ref[2:3, :], 0.0, w)
    cy2 = jnp.clip(bx_ref[3:4, :], 0.0, h)
    valid = ((cx2 - cx1) >= MIN_SIZE) & ((cy2 - cy1) >= MIN_SIZE)
    out_ref[...] = jnp.where(valid, sc_ref[...], -1.0)


def _nms_kernel(hw_ref, g_ref, gc_ref, sv_ref, out_ref, gclip_ref, m_ref):
    h = hw_ref[0]
    w = hw_ref[1]

    cx1r = jnp.clip(g_ref[:, 0:1], 0.0, w)
    cy1r = jnp.clip(g_ref[:, 1:2], 0.0, h)
    cx2r = jnp.clip(g_ref[:, 2:3], 0.0, w)
    cy2r = jnp.clip(g_ref[:, 3:4], 0.0, h)
    gclip_ref[:, 0:1] = cx1r
    gclip_ref[:, 1:2] = cy1r
    gclip_ref[:, 2:3] = cx2r
    gclip_ref[:, 3:4] = cy2r
    gclip_ref[:, 4:5] = g_ref[:, 4:5]
    gclip_ref[:, 5:6] = (cx2r - cx1r) * (cy2r - cy1r)
    gclip_ref[:, 6:8] = jnp.zeros((K_PAD, 2), jnp.float32)

    cx1c = jnp.clip(gc_ref[0:1, :], 0.0, w)
    cy1c = jnp.clip(gc_ref[1:2, :], 0.0, h)
    cx2c = jnp.clip(gc_ref[2:3, :], 0.0, w)
    cy2c = jnp.clip(gc_ref[3:4, :], 0.0, h)
    areas_c = (cx2c - cx1c) * (cy2c - cy1c)

    col_j = lax.broadcasted_iota(jnp.int32, (ROW_BLK, K_PAD), 1)
    row_i0 = lax.broadcasted_iota(jnp.int32, (ROW_BLK, K_PAD), 0)

    def make_block(i, carry):
        r0 = i * ROW_BLK
        x1b = gclip_ref[pl.ds(r0, ROW_BLK), 0:1]
        y1b = gclip_ref[pl.ds(r0, ROW_BLK), 1:2]
        x2b = gclip_ref[pl.ds(r0, ROW_BLK), 2:3]
        y2b = gclip_ref[pl.ds(r0, ROW_BLK), 3:4]
        ab = gclip_ref[pl.ds(r0, ROW_BLK), 5:6]
        xx1 = jnp.maximum(x1b, cx1c)
        yy1 = jnp.maximum(y1b, cy1c)
        xx2 = jnp.minimum(x2b, cx2c)
        yy2 = jnp.minimum(y2b, cy2c)
        inter = jnp.maximum(xx2 - xx1, 0.0) * jnp.maximum(yy2 - yy1, 0.0)
        iou = inter / (ab + areas_c - inter + 1e-06)
        upper = col_j > (row_i0 + r0)
        m_ref[pl.ds(r0, ROW_BLK), :] = ((iou > IOU_THR) & upper).astype(
            jnp.float32
        )
        return carry

    lax.fori_loop(0, K_PAD // ROW_BLK, make_block, 0)

    init_sup = (sv_ref[...] <= -0.5).astype(jnp.float32)

    def cond_fn(carry):
        return carry[1]

    def body_fn(carry):
        s, _ = carry
        kept = 1.0 - s
        contrib = jnp.dot(
            kept, m_ref[...], preferred_element_type=jnp.float32
        )
        s_new = jnp.maximum(init_sup, (contrib > 0.0).astype(jnp.float32))
        changed = jnp.any(s_new != s)
        return (s_new, changed)

    s_fix, _ = lax.while_loop(
        cond_fn, body_fn, (init_sup, jnp.asarray(True))
    )

    kept = 1.0 - s_fix

    posq = kept
    shift = 1
    while shift < K_PAD:
        shifted = jnp.concatenate(
            [jnp.zeros((1, shift), jnp.float32), posq[:, : K_PAD - shift]],
            axis=1,
        )
        posq = posq + shifted
        shift *= 2

    p_iota = lax.broadcasted_iota(jnp.int32, (OUT_ROWS, K_PAD), 0) + 1
    onehot = jnp.where(
        (p_iota == posq.astype(jnp.int32)) & (kept > 0.0), 1.0, 0.0
    )
    out_ref[...] = jnp.dot(
        onehot, gclip_ref[...], preferred_element_type=jnp.float32
    )


def _run_mask(hw, bx, sc):
    return pl.pallas_call(
        _mask_scores_kernel,
        out_shape=jax.ShapeDtypeStruct((1, N_PAD), jnp.float32),
        in_specs=[
            pl.BlockSpec(memory_space=pltpu.SMEM),
            pl.BlockSpec(memory_space=pltpu.VMEM),
            pl.BlockSpec(memory_space=pltpu.VMEM),
        ],
        out_specs=pl.BlockSpec(memory_space=pltpu.VMEM),
    )(hw, bx, sc)


def _run_nms(hw, g, gc, sv):
    return pl.pallas_call(
        _nms_kernel,
        out_shape=jax.ShapeDtypeStruct((OUT_ROWS, 8), jnp.float32),
        in_specs=[
            pl.BlockSpec(memory_space=pltpu.SMEM),
            pl.BlockSpec(memory_space=pltpu.VMEM),
            pl.BlockSpec(memory_space=pltpu.VMEM),
            pl.BlockSpec(memory_space=pltpu.VMEM),
        ],
        out_specs=pl.BlockSpec(memory_space=pltpu.VMEM),
        scratch_shapes=[
            pltpu.VMEM((K_PAD, 8), jnp.float32),
            pltpu.VMEM((K_PAD, K_PAD), jnp.float32),
        ],
    )(hw, g, gc, sv)


def kernel(boxes, scores, img_h, img_w):
    h = jnp.asarray(img_h, jnp.float32)
    w = jnp.asarray(img_w, jnp.float32)
    hw = jnp.stack([h, w])

    bx = jnp.pad(boxes, ((0, N_PAD - N_BOX), (0, 0))).T
    sc = jnp.pad(scores, (0, N_PAD - N_BOX))[None, :]

    masked = _run_mask(hw, bx, sc)
    vals, idx = lax.top_k(masked[0], K_SEL)

    table = jnp.concatenate([boxes, scores[:, None]], axis=1)
    g = jnp.take(table, idx, axis=0)
    g = jnp.pad(g, ((0, K_PAD - K_SEL), (0, 3)))
    gc = g.T
    sv = jnp.pad(vals, (0, K_PAD - K_SEL), constant_values=-1.0)[None, :]

    out = _run_nms(hw, g, gc, sv)
    return out[:POST, :5]
